# baseline (speedup 1.0000x reference)
import functools

import jax
import jax.numpy as jnp
from jax import lax
from jax.experimental import pallas as pl
from jax.experimental.pallas import tpu as pltpu
from jax.experimental.pallas import tpu_sc as plsc

START_IND = 1024
END_IND = 1025

B = 8
LT = 130
TP = 256
TC = 514
NC = 8
D = 1024
V = 1026
VR = 1024
S = 900
DC = D * NC

_NW = 32


def _sc_body(text_ids, prom_ids, code_ids, ttab, wflat, pos, mvec,
             text_e, prom_f, code_f, mask_f,
             idx16, rows16, pos2, out2, trows, tpos, mall, mflat, sem):
    cid = lax.axis_index("c")
    sid = lax.axis_index("s")
    wid = sid * 2 + cid

    lanes = lax.iota(jnp.int32, 16)
    lane_lo = lanes < 8
    offs = (lanes & 7) * V
    col0 = lanes >> 3
    row_t0 = lanes & 7
    row_t1 = (lanes & 7) + 8

    def text_chunk(c, t0, n):
        b = c >> 3
        pltpu.sync_copy(text_ids.at[b, pl.ds(t0, 16)], idx16)
        pltpu.async_copy(ttab.at[idx16], trows, sem).wait()
        pltpu.sync_copy(pos.at[pl.ds(t0, 16), :], tpos)

        def add_body(j, _):
            r = j >> 6
            cc = (j & 63) * 16
            trows[r, pl.ds(cc, 16)] = trows[r, pl.ds(cc, 16)] + tpos[r, pl.ds(cc, 16)]
            return 0

        lax.fori_loop(0, n * 64, add_body, 0)
        if n == 16:
            pltpu.sync_copy(trows, text_e.at[b, pl.ds(t0, 16), :])
        else:
            pltpu.sync_copy(trows.at[pl.ds(0, 2), :], text_e.at[b, pl.ds(t0, 2), :])

    def text_loop(k, _):
        c = wid + k * 32
        t0 = (c & 7) * 16
        text_chunk(c, t0, 16)
        return 0

    lax.fori_loop(0, 2, text_loop, 0)

    @pl.when(wid < 8)
    def _():
        text_chunk(wid * 8, 128, 2)

    def wave_pair(ids, out, T, p):
        npairs_b = T // 2
        b = p // npairs_b
        q = p - b * npairs_b
        t = q * 2
        pltpu.sync_copy(ids.at[b, pl.ds(t * 8, 16)], idx16)
        iv = idx16[...] + offs
        idx16[...] = iv
        gather = pltpu.async_copy(wflat.at[idx16], rows16, sem)
        pltpu.sync_copy(pos.at[pl.ds(t, 2), :], pos2)
        gather.wait()

        def ileave(jj, _):
            for tt, rowv in ((0, row_t0), (1, row_t1)):
                pv16 = pos2[tt, pl.ds(jj * 16, 16)]
                for k in range(8):
                    j = jj * 8 + k
                    col = col0 + 2 * j
                    g = plsc.load_gather(rows16, [rowv, col])
                    pv = jnp.where(
                        lane_lo,
                        jnp.full((16,), pv16[2 * k], jnp.float32),
                        jnp.full((16,), pv16[2 * k + 1], jnp.float32))
                    out2[tt, pl.ds(j * 16, 16)] = g + pv
            return 0

        lax.fori_loop(0, 64, ileave, 0)
        pltpu.sync_copy(out2, out.at[b, pl.ds(t, 2), :])

    def prom_loop(k, _):
        wave_pair(prom_ids, prom_f, TP, wid + k * 32)
        return 0

    lax.fori_loop(0, (B * (TP // 2)) // 32, prom_loop, 0)

    ncode = B * (TC // 2)
    ncode_mine = jnp.where(wid < ncode - 32 * (ncode // 32), ncode // 32 + 1,
                           ncode // 32)

    def code_loop(k, _):
        wave_pair(code_ids, code_f, TC, wid + k * 32)
        return 0

    lax.fori_loop(0, ncode_mine, code_loop, 0)

    pltpu.sync_copy(mvec, mall)
    nmask = (B * S * S) // 3600
    nmask_mine = jnp.where(wid < nmask - 32 * (nmask // 32), nmask // 32 + 1,
                           nmask // 32)

    def mask_chunk(k, _):
        ch = wid + k * 32
        base = ch * 3600

        def mvreg(u, _):
            flat = base + u * 16 + lanes
            r_g = flat // (S * S)
            rem = flat - r_g * (S * S)
            rr = rem // S
            cc = rem - rr * S
            a = plsc.load_gather(mall, [r_g, rr])
            bb = plsc.load_gather(mall, [r_g, cc])
            mflat[pl.ds(u * 16, 16)] = a * bb
            return 0

        lax.fori_loop(0, 225, mvreg, 0)
        pltpu.sync_copy(mflat, mask_f.at[pl.ds(base, 3600)])
        return 0

    lax.fori_loop(0, nmask_mine, mask_chunk, 0)


@jax.jit
def _sc_call(text_ids_pad, prom_ids_f, code_ids_f, text_table, wflat, pos,
             mvec):
    mesh = plsc.VectorSubcoreMesh(core_axis_name="c", subcore_axis_name="s",
                                  num_cores=2, num_subcores=16)
    f = pl.kernel(
        _sc_body,
        mesh=mesh,
        compiler_params=pltpu.CompilerParams(use_tc_tiling_on_sc=False,
                                             needs_layout_passes=False),
        out_type=[
            jax.ShapeDtypeStruct((B, LT, D), jnp.float32),
            jax.ShapeDtypeStruct((B, TP, DC), jnp.float32),
            jax.ShapeDtypeStruct((B, TC, DC), jnp.float32),
            jax.ShapeDtypeStruct((B * S * S,), jnp.float32),
        ],
        scratch_types=[
            pltpu.VMEM((16,), jnp.int32),
            pltpu.VMEM((16, VR), jnp.float32),
            pltpu.VMEM((2, D), jnp.float32),
            pltpu.VMEM((2, DC), jnp.float32),
            pltpu.VMEM((16, D), jnp.float32),
            pltpu.VMEM((16, D), jnp.float32),
            pltpu.VMEM((B, 1024), jnp.float32),
            pltpu.VMEM((3600,), jnp.float32),
            pltpu.SemaphoreType.DMA,
        ],
    )
    return f(text_ids_pad, prom_ids_f, code_ids_f, text_table, wflat, pos,
             mvec)


def kernel(text, prom, code, text_table, wave_tables, pos_emb):
    text = jnp.where(text == -1, END_IND, text)
    prom_ids = jnp.where(prom == -1, END_IND, prom)
    code_in = jnp.where(code == -1, END_IND, code)
    text_ids = jnp.pad(text, ((0, 0), (1, 0)), constant_values=START_IND)
    text_ids = jnp.pad(text_ids, ((0, 0), (0, 1)), constant_values=END_IND)
    code_ids = jnp.pad(code_in, ((0, 0), (1, 0), (0, 0)),
                       constant_values=START_IND)
    code_ids = jnp.pad(code_ids, ((0, 0), (0, 1), (0, 0)),
                       constant_values=END_IND)
    gt = code_ids

    text_mask = jnp.pad((text_ids != END_IND)[:, :-1], ((0, 0), (1, 0)),
                        constant_values=True)
    prom_mask = jnp.pad((prom_ids != END_IND)[:, :-1, 0], ((0, 0), (1, 0)),
                        constant_values=True)
    code_mask = jnp.pad((code_ids != END_IND)[:, :-1, 0], ((0, 0), (1, 0)),
                        constant_values=True)
    m = jnp.concatenate((text_mask, prom_mask, code_mask), axis=1
                        ).astype(jnp.float32)
    gt_mask = code_mask

    text_ids_pad = jnp.pad(text_ids, ((0, 0), (0, 144 - LT)))
    prom_ids_f = prom_ids.reshape(B, TP * NC)
    code_ids_f = code_ids.reshape(B, TC * NC)
    wflat = wave_tables.reshape(NC * V, VR)
    mvec = jnp.pad(m, ((0, 0), (0, 1024 - S)))

    text_e, prom_f, code_f, mask_f = _sc_call(
        text_ids_pad, prom_ids_f, code_ids_f, text_table, wflat,
        pos_emb[:TC], mvec)

    prom_e = prom_f.reshape(B, TP, D, NC)
    code_e = code_f.reshape(B, TC, D, NC)
    mask = mask_f.reshape(B, S, S)
    return (text_e, prom_e, code_e, mask, gt, gt_mask)


# speedup vs baseline: 1.4768x; 1.4768x over previous
import functools

import jax
import jax.numpy as jnp
from jax import lax
from jax.experimental import pallas as pl
from jax.experimental.pallas import tpu as pltpu
from jax.experimental.pallas import tpu_sc as plsc

START_IND = 1024
END_IND = 1025

B = 8
LT = 130
TP = 256
TC = 514
NC = 8
D = 1024
V = 1026
VR = 1024
S = 900
DC = D * NC

_NW = 32


def _sc_body(text_ids, prom_ids, code_ids, ttab, wflat, pos, mvec,
             text_e, prom_f, code_f, mask_f,
             g0, g1, o0, o1, p0, p1, i0, i1, idsbuf, mall, mflat,
             sg0, sg1, sp0, sp1, so0, so1, st):
    cid = lax.axis_index("c")
    sid = lax.axis_index("s")
    wid = sid * 2 + cid

    lanes = lax.iota(jnp.int32, 16)
    lane_lo = lanes < 8
    offs = (lanes & 7) * V
    col0 = lanes >> 3
    row_t0 = lanes & 7
    row_t1 = (lanes & 7) + 8

    gb = (g0, g1)
    ob = (o0, o1)
    pb = (p0, p1)
    ib = (i0, i1)
    sgb = (sg0, sg1)
    spb = (sp0, sp1)
    sob = (so0, so1)

    def text_chunk(b, t0, n):
        pltpu.sync_copy(text_ids.at[b, pl.ds(t0, 16)], i0)
        pltpu.async_copy(ttab.at[i0], g0, st).wait()
        pltpu.sync_copy(pos.at[pl.ds(t0, 16), :], g1)

        def add_body(j, _):
            r = j >> 6
            cc = (j & 63) * 16
            g0[r, pl.ds(cc, 16)] = g0[r, pl.ds(cc, 16)] + g1[r, pl.ds(cc, 16)]
            return 0

        lax.fori_loop(0, n * 64, add_body, 0)
        if n == 16:
            pltpu.sync_copy(g0, text_e.at[b, pl.ds(t0, 16), :])
        else:
            pltpu.sync_copy(g0.at[pl.ds(0, 2), :], text_e.at[b, pl.ds(t0, 2), :])

    def text_loop(k, _):
        c = wid + k * 32
        text_chunk(c >> 3, (c & 7) * 16, 16)
        return 0

    lax.fori_loop(0, 2, text_loop, 0)

    @pl.when(wid < 8)
    def _():
        text_chunk(wid, 128, 2)

    def segment(ids_flat, out, T, start, n, nids):
        npb = T // 2
        n = jnp.int32(n)

        def pair_coords(k):
            p = start + k
            b = p // npb
            t = 2 * (p - b * npb)
            return b, t

        pltpu.sync_copy(ids_flat.at[pl.ds(16 * start, nids)],
                        idsbuf.at[pl.ds(0, nids)])

        def issue(k, par):
            b, t = pair_coords(k)
            ib[par][...] = idsbuf[pl.ds(16 * k, 16)] + offs
            pltpu.async_copy(wflat.at[ib[par]], gb[par], sgb[par])
            pltpu.async_copy(pos.at[pl.ds(t, 2), :], pb[par], spb[par])

        def process(k, par):
            b, t = pair_coords(k)
            pltpu.make_async_copy(wflat.at[ib[par]], gb[par], sgb[par]).wait()
            pltpu.make_async_copy(pos.at[pl.ds(t, 2), :], pb[par],
                                  spb[par]).wait()
            rows = gb[par]
            pvs = pb[par]
            out2 = ob[par]

            def ileave(jj, _):
                for tt, rowv in ((0, row_t0), (1, row_t1)):
                    pv16 = pvs[tt, pl.ds(jj * 16, 16)]
                    for kk in range(8):
                        j = jj * 8 + kk
                        col = col0 + 2 * j
                        g = plsc.load_gather(rows, [rowv, col])
                        pv = jnp.where(
                            lane_lo,
                            jnp.full((16,), pv16[2 * kk], jnp.float32),
                            jnp.full((16,), pv16[2 * kk + 1], jnp.float32))
                        out2[tt, pl.ds(j * 16, 16)] = g + pv
                return 0

            lax.fori_loop(0, 64, ileave, 0)
            pltpu.async_copy(out2, out.at[b, pl.ds(t, 2), :], sob[par])

        def drain_out(k, par):
            b, t = pair_coords(k)
            pltpu.make_async_copy(ob[par], out.at[b, pl.ds(t, 2), :],
                                  sob[par]).wait()

        @pl.when(n > 0)
        def _():
            issue(0, 0)

        def outer(kk, _):
            for par in range(2):
                k = kk * 2 + par

                @pl.when(k < n)
                def _():
                    @pl.when(k + 1 < n)
                    def _():
                        issue(k + 1, 1 - par)

                    @pl.when(k >= 2)
                    def _():
                        drain_out(k - 2, par)

                    process(k, par)
            return 0

        lax.fori_loop(0, (n + 1) // 2, outer, 0)

        for par in range(2):
            klast = jnp.where((n - 1) % 2 == par, n - 1, n - 2)

            @pl.when(klast >= 0)
            def _(klast=klast, par=par):
                drain_out(klast, par)

    segment(prom_ids, prom_f, TP, wid * 32, 32, 512)
    ncode = B * (TC // 2)
    nc_extra = ncode - 32 * (ncode // 32)
    code_start = wid * (ncode // 32) + jnp.minimum(wid, nc_extra)
    code_n = ncode // 32 + jnp.where(wid < nc_extra, 1, 0)
    segment(code_ids, code_f, TC, code_start, code_n, 1056)

    pltpu.sync_copy(mvec, mall)
    nmask = (B * S * S) // 3600
    nmask_mine = jnp.where(wid < nmask - 32 * (nmask // 32), nmask // 32 + 1,
                           nmask // 32)
    lane4 = lanes & 3

    def mask_chunk(k, _):
        ch = wid + k * 32
        a4 = ch * 4
        base = ch * 3600
        gr = a4 + lane4
        bv = gr // S
        rv = gr - bv * S
        mv16 = plsc.load_gather(mall, [bv, rv])
        s0 = jnp.full((16,), mv16[0], jnp.float32)
        s1 = jnp.full((16,), mv16[1], jnp.float32)
        s2 = jnp.full((16,), mv16[2], jnp.float32)
        s3 = jnp.full((16,), mv16[3], jnp.float32)
        b0 = a4 // S

        def mvreg(u, _):
            fl = u * 16 + lanes
            rowv = (jnp.where(fl >= S, 1, 0) + jnp.where(fl >= 2 * S, 1, 0)
                    + jnp.where(fl >= 3 * S, 1, 0))
            colv = fl - rowv * S
            blv = jnp.where(a4 + rowv >= (b0 + 1) * S, b0 + 1, b0)
            av = jnp.where(fl < S, s0,
                           jnp.where(fl < 2 * S, s1,
                                     jnp.where(fl < 3 * S, s2, s3)))
            bb = plsc.load_gather(mall, [blv, colv])
            mflat[pl.ds(u * 16, 16)] = av * bb
            return 0

        lax.fori_loop(0, 225, mvreg, 0)
        pltpu.sync_copy(mflat, mask_f.at[pl.ds(base, 3600)])
        return 0

    lax.fori_loop(0, nmask_mine, mask_chunk, 0)


@jax.jit
def _sc_call(text_ids_pad, prom_ids_f, code_ids_f, text_table, wflat, pos,
             mvec):
    mesh = plsc.VectorSubcoreMesh(core_axis_name="c", subcore_axis_name="s",
                                  num_cores=2, num_subcores=16)
    f = pl.kernel(
        _sc_body,
        mesh=mesh,
        compiler_params=pltpu.CompilerParams(use_tc_tiling_on_sc=False,
                                             needs_layout_passes=False),
        out_type=[
            jax.ShapeDtypeStruct((B, LT, D), jnp.float32),
            jax.ShapeDtypeStruct((B, TP, DC), jnp.float32),
            jax.ShapeDtypeStruct((B, TC, DC), jnp.float32),
            jax.ShapeDtypeStruct((B * S * S,), jnp.float32),
        ],
        scratch_types=[
            pltpu.VMEM((16, VR), jnp.float32),
            pltpu.VMEM((16, VR), jnp.float32),
            pltpu.VMEM((2, DC), jnp.float32),
            pltpu.VMEM((2, DC), jnp.float32),
            pltpu.VMEM((2, D), jnp.float32),
            pltpu.VMEM((2, D), jnp.float32),
            pltpu.VMEM((16,), jnp.int32),
            pltpu.VMEM((16,), jnp.int32),
            pltpu.VMEM((1056,), jnp.int32),
            pltpu.VMEM((B, 1024), jnp.float32),
            pltpu.VMEM((3600,), jnp.float32),
            pltpu.SemaphoreType.DMA,
            pltpu.SemaphoreType.DMA,
            pltpu.SemaphoreType.DMA,
            pltpu.SemaphoreType.DMA,
            pltpu.SemaphoreType.DMA,
            pltpu.SemaphoreType.DMA,
            pltpu.SemaphoreType.DMA,
        ],
    )
    return f(text_ids_pad, prom_ids_f, code_ids_f, text_table, wflat, pos,
             mvec)


def kernel(text, prom, code, text_table, wave_tables, pos_emb):
    text = jnp.where(text == -1, END_IND, text)
    prom_ids = jnp.where(prom == -1, END_IND, prom)
    code_in = jnp.where(code == -1, END_IND, code)
    text_ids = jnp.pad(text, ((0, 0), (1, 0)), constant_values=START_IND)
    text_ids = jnp.pad(text_ids, ((0, 0), (0, 1)), constant_values=END_IND)
    code_ids = jnp.pad(code_in, ((0, 0), (1, 0), (0, 0)),
                       constant_values=START_IND)
    code_ids = jnp.pad(code_ids, ((0, 0), (0, 1), (0, 0)),
                       constant_values=END_IND)
    gt = code_ids

    text_mask = jnp.pad((text_ids != END_IND)[:, :-1], ((0, 0), (1, 0)),
                        constant_values=True)
    prom_mask = jnp.pad((prom_ids != END_IND)[:, :-1, 0], ((0, 0), (1, 0)),
                        constant_values=True)
    code_mask = jnp.pad((code_ids != END_IND)[:, :-1, 0], ((0, 0), (1, 0)),
                        constant_values=True)
    m = jnp.concatenate((text_mask, prom_mask, code_mask), axis=1
                        ).astype(jnp.float32)
    gt_mask = code_mask

    text_ids_pad = jnp.pad(text_ids, ((0, 0), (0, 144 - LT)))
    prom_ids_f = prom_ids.reshape(B * TP * NC)
    code_ids_f = jnp.pad(code_ids.reshape(B * TC * NC), (0, 33024 - B * TC * NC))
    wflat = wave_tables.reshape(NC * V, VR)
    mvec = jnp.pad(m, ((0, 0), (0, 1024 - S)))

    text_e, prom_f, code_f, mask_f = _sc_call(
        text_ids_pad, prom_ids_f, code_ids_f, text_table, wflat,
        pos_emb[:TC], mvec)

    prom_e = prom_f.reshape(B, TP, D, NC)
    code_e = code_f.reshape(B, TC, D, NC)
    mask = mask_f.reshape(B, S, S)
    return (text_e, prom_e, code_e, mask, gt, gt_mask)


# speedup vs baseline: 4.8885x; 3.3102x over previous
import functools

import jax
import jax.numpy as jnp
from jax import lax
from jax.experimental import pallas as pl
from jax.experimental.pallas import tpu as pltpu
from jax.experimental.pallas import tpu_sc as plsc

START_IND = 1024
END_IND = 1025

B = 8
LT = 130
TP = 256
TC = 514
NC = 8
D = 1024
V = 1026
S = 900

_NW = 32
_NB = 3


def _sc_body(text_ids, prom_ids, code_ids, ttab, wflat, pos, mvec,
             text_e, prom_g, code_g, mask_f,
             g0, g1, g2, p0, p1, p2, i0, i1, i2, idsbuf, mall, mflat,
             sg0, sg1, sg2, sp0, sp1, sp2, so0, so1, so2, st):
    cid = lax.axis_index("c")
    sid = lax.axis_index("s")
    wid = sid * 2 + cid

    lanes = lax.iota(jnp.int32, 16)
    offs8 = lanes & 7

    gb = (g0, g1, g2)
    pb = (p0, p1, p2)
    ib = (i0, i1, i2)
    sgb = (sg0, sg1, sg2)
    spb = (sp0, sp1, sp2)
    sob = (so0, so1, so2)

    def text_chunk(b, t0, n):
        pltpu.sync_copy(text_ids.at[b, pl.ds(t0, 16)], i0)
        pltpu.async_copy(ttab.at[i0], g0, st).wait()
        pltpu.sync_copy(pos.at[pl.ds(t0, 16), :], g1)

        def add_body(j, _):
            r = j >> 6
            cc = (j & 63) * 16
            g0[r, pl.ds(cc, 16)] = g0[r, pl.ds(cc, 16)] + g1[r, pl.ds(cc, 16)]
            return 0

        lax.fori_loop(0, n * 64, add_body, 0)
        if n == 16:
            pltpu.sync_copy(g0, text_e.at[b, pl.ds(t0, 16), :])
        else:
            pltpu.sync_copy(g0.at[pl.ds(0, 2), :], text_e.at[b, pl.ds(t0, 2), :])

    def text_loop(k, _):
        c = wid + k * 32
        text_chunk(c >> 3, (c & 7) * 16, 16)
        return 0

    lax.fori_loop(0, 2, text_loop, 0)

    @pl.when(wid < 8)
    def _():
        text_chunk(wid, 128, 2)

    def segment(ids_flat, out, T, start, n, nids):
        npb = T // 2
        n = jnp.int32(n)

        def pair_coords(k):
            p = start + k
            b = p // npb
            t = 2 * (p - b * npb)
            return b, t

        pltpu.sync_copy(ids_flat.at[pl.ds(16 * start, nids)],
                        idsbuf.at[pl.ds(0, nids)])

        def issue(k, par):
            b, t = pair_coords(k)
            ib[par][...] = (idsbuf[pl.ds(16 * k, 16)] << 3) + offs8
            pltpu.async_copy(wflat.at[ib[par]], gb[par], sgb[par])
            pltpu.async_copy(pos.at[pl.ds(t, 2), :], pb[par], spb[par])

        def drain_out(k, par):
            b, t = pair_coords(k)
            pltpu.make_async_copy(gb[par], out.at[b, pl.ds(t * 8, 16), :],
                                  sob[par]).wait()

        def process(k, par):
            b, t = pair_coords(k)
            pltpu.make_async_copy(wflat.at[ib[par]], gb[par], sgb[par]).wait()
            pltpu.make_async_copy(pos.at[pl.ds(t, 2), :], pb[par],
                                  spb[par]).wait()
            rows = gb[par]
            pvs = pb[par]

            def addp(c, _):
                cc = c * 16
                for tt in range(2):
                    pv = pvs[tt, pl.ds(cc, 16)]
                    for i in range(8):
                        r = 8 * tt + i
                        rows[r, pl.ds(cc, 16)] = rows[r, pl.ds(cc, 16)] + pv
                return 0

            lax.fori_loop(0, 64, addp, 0)
            pltpu.async_copy(rows, out.at[b, pl.ds(t * 8, 16), :], sob[par])

        @pl.when(n > 0)
        def _():
            issue(0, 0)

        def outer(kk, _):
            for par in range(_NB):
                k = kk * _NB + par

                @pl.when(k < n)
                def _(k=k, par=par):
                    kn = k + 1
                    parn = (par + 1) % _NB

                    @pl.when(kn < n)
                    def _():
                        @pl.when(kn >= _NB)
                        def _():
                            drain_out(kn - _NB, parn)

                        issue(kn, parn)

                    process(k, par)
            return 0

        lax.fori_loop(0, (n + _NB - 1) // _NB, outer, 0)

        for par in range(_NB):
            klast = jnp.where((n - 1) % _NB == par, n - 1,
                              jnp.where((n - 2) % _NB == par, n - 2, n - 3))

            @pl.when(klast >= 0)
            def _(klast=klast, par=par):
                drain_out(klast, par)

    segment(prom_ids, prom_g, TP, wid * 32, 32, 512)
    ncode = B * (TC // 2)
    nc_extra = ncode - 32 * (ncode // 32)
    code_start = wid * (ncode // 32) + jnp.minimum(wid, nc_extra)
    code_n = ncode // 32 + jnp.where(wid < nc_extra, 1, 0)
    segment(code_ids, code_g, TC, code_start, code_n, 1056)

    pltpu.sync_copy(mvec, mall)
    nmask = (B * S * S) // 3600
    nmask_mine = jnp.where(wid < nmask - 32 * (nmask // 32), nmask // 32 + 1,
                           nmask // 32)
    lane4 = lanes & 3

    def mask_chunk(k, _):
        ch = wid + k * 32
        a4 = ch * 4
        base = ch * 3600
        gr = a4 + lane4
        bv = gr // S
        rv = gr - bv * S
        mv16 = plsc.load_gather(mall, [bv, rv])
        s0 = jnp.full((16,), mv16[0], jnp.float32)
        s1 = jnp.full((16,), mv16[1], jnp.float32)
        s2 = jnp.full((16,), mv16[2], jnp.float32)
        s3 = jnp.full((16,), mv16[3], jnp.float32)
        b0 = a4 // S

        def mvreg(u, _):
            fl = u * 16 + lanes
            rowv = (jnp.where(fl >= S, 1, 0) + jnp.where(fl >= 2 * S, 1, 0)
                    + jnp.where(fl >= 3 * S, 1, 0))
            colv = fl - rowv * S
            blv = jnp.where(a4 + rowv >= (b0 + 1) * S, b0 + 1, b0)
            av = jnp.where(fl < S, s0,
                           jnp.where(fl < 2 * S, s1,
                                     jnp.where(fl < 3 * S, s2, s3)))
            bb = plsc.load_gather(mall, [blv, colv])
            mflat[pl.ds(u * 16, 16)] = av * bb
            return 0

        lax.fori_loop(0, 225, mvreg, 0)
        pltpu.sync_copy(mflat, mask_f.at[pl.ds(base, 3600)])
        return 0

    lax.fori_loop(0, nmask_mine, mask_chunk, 0)


@jax.jit
def _sc_call(text_ids_pad, prom_ids_f, code_ids_f, text_table, wflat, pos,
             mvec):
    mesh = plsc.VectorSubcoreMesh(core_axis_name="c", subcore_axis_name="s",
                                  num_cores=2, num_subcores=16)
    f = pl.kernel(
        _sc_body,
        mesh=mesh,
        compiler_params=pltpu.CompilerParams(use_tc_tiling_on_sc=False,
                                             needs_layout_passes=False),
        out_type=[
            jax.ShapeDtypeStruct((B, LT, D), jnp.float32),
            jax.ShapeDtypeStruct((B, TP * NC, D), jnp.float32),
            jax.ShapeDtypeStruct((B, TC * NC, D), jnp.float32),
            jax.ShapeDtypeStruct((B * S * S,), jnp.float32),
        ],
        scratch_types=[
            pltpu.VMEM((16, D), jnp.float32),
            pltpu.VMEM((16, D), jnp.float32),
            pltpu.VMEM((16, D), jnp.float32),
            pltpu.VMEM((2, D), jnp.float32),
            pltpu.VMEM((2, D), jnp.float32),
            pltpu.VMEM((2, D), jnp.float32),
            pltpu.VMEM((16,), jnp.int32),
            pltpu.VMEM((16,), jnp.int32),
            pltpu.VMEM((16,), jnp.int32),
            pltpu.VMEM((1056,), jnp.int32),
            pltpu.VMEM((B, 1024), jnp.float32),
            pltpu.VMEM((3600,), jnp.float32),
            pltpu.SemaphoreType.DMA,
            pltpu.SemaphoreType.DMA,
            pltpu.SemaphoreType.DMA,
            pltpu.SemaphoreType.DMA,
            pltpu.SemaphoreType.DMA,
            pltpu.SemaphoreType.DMA,
            pltpu.SemaphoreType.DMA,
            pltpu.SemaphoreType.DMA,
            pltpu.SemaphoreType.DMA,
            pltpu.SemaphoreType.DMA,
        ],
    )
    return f(text_ids_pad, prom_ids_f, code_ids_f, text_table, wflat, pos,
             mvec)


def kernel(text, prom, code, text_table, wave_tables, pos_emb):
    text = jnp.where(text == -1, END_IND, text)
    prom_ids = jnp.where(prom == -1, END_IND, prom)
    code_in = jnp.where(code == -1, END_IND, code)
    text_ids = jnp.pad(text, ((0, 0), (1, 0)), constant_values=START_IND)
    text_ids = jnp.pad(text_ids, ((0, 0), (0, 1)), constant_values=END_IND)
    code_ids = jnp.pad(code_in, ((0, 0), (1, 0), (0, 0)),
                       constant_values=START_IND)
    code_ids = jnp.pad(code_ids, ((0, 0), (0, 1), (0, 0)),
                       constant_values=END_IND)
    gt = code_ids

    text_mask = jnp.pad((text_ids != END_IND)[:, :-1], ((0, 0), (1, 0)),
                        constant_values=True)
    prom_mask = jnp.pad((prom_ids != END_IND)[:, :-1, 0], ((0, 0), (1, 0)),
                        constant_values=True)
    code_mask = jnp.pad((code_ids != END_IND)[:, :-1, 0], ((0, 0), (1, 0)),
                        constant_values=True)
    m = jnp.concatenate((text_mask, prom_mask, code_mask), axis=1
                        ).astype(jnp.float32)
    gt_mask = code_mask

    text_ids_pad = jnp.pad(text_ids, ((0, 0), (0, 144 - LT)))
    prom_ids_f = prom_ids.reshape(B * TP * NC)
    code_ids_f = jnp.pad(code_ids.reshape(B * TC * NC), (0, 33024 - B * TC * NC))
    wflat = jnp.transpose(wave_tables, (1, 0, 2)).reshape(V * NC, D)
    mvec = jnp.pad(m, ((0, 0), (0, 1024 - S)))

    text_e, prom_g, code_g, mask_f = _sc_call(
        text_ids_pad, prom_ids_f, code_ids_f, text_table, wflat,
        pos_emb[:TC + 6], mvec)

    prom_e = prom_g.reshape(B, TP, NC, D).transpose(0, 1, 3, 2)
    code_e = code_g.reshape(B, TC, NC, D).transpose(0, 1, 3, 2)
    mask = mask_f.reshape(B, S, S)
    return (text_e, prom_e, code_e, mask, gt, gt_mask)
